# baseline (speedup 1.0000x reference)
import functools
import jax
import jax.numpy as jnp
from jax import lax
from jax.experimental import pallas as pl
from jax.experimental.pallas import tpu as pltpu
from jax.experimental.pallas import tpu_sc as plsc

N = 10000
NP = 10240
E = 320000
XDIM = 128
HEADS = 8
HDIM = 64
G = 64
HOUT = 512

NC = 2
NS = 16
MBLK = 640
KA = 80
KB = 64
NBUF = 5
EP = 327680
ECA = E // (NC * NS)
ECB = EP // NS
CH = 5120
RPT = NP // NS

_mesh = plsc.VectorSubcoreMesh(core_axis_name="c", subcore_axis_name="s")

_GDN = lax.GatherDimensionNumbers(
    offset_dims=(), collapsed_slice_dims=(0,), start_index_map=(0,))


def _take16(v, idx):
    return lax.gather(v, idx[:, None], _GDN, (1,),
                      mode=lax.GatherScatterMode.PROMISE_IN_BOUNDS)


def _mm_kernel(x_ref, w_ref, o_ref):
    o_ref[...] = jnp.dot(x_ref[...], w_ref[...], preferred_element_type=jnp.float32)


def _matmul(x, w):
    m, k = x.shape
    _, n = w.shape
    return pl.pallas_call(
        _mm_kernel,
        grid=(m // MBLK,),
        in_specs=[
            pl.BlockSpec((MBLK, k), lambda i: (i, 0)),
            pl.BlockSpec((k, n), lambda i: (0, 0)),
        ],
        out_specs=pl.BlockSpec((MBLK, n), lambda i: (i, 0)),
        out_shape=jax.ShapeDtypeStruct((m, n), jnp.float32),
    )(x, w)


def _mm_h2_kernel(x_ref, w_ref, o_ref):
    o_ref[0] = jnp.dot(x_ref[...], w_ref[...], preferred_element_type=jnp.float32)


def _matmul_h2(x, w):
    m, k = x.shape
    return pl.pallas_call(
        _mm_h2_kernel,
        grid=(4, m // MBLK),
        in_specs=[
            pl.BlockSpec((MBLK, k), lambda q, i: (i, 0)),
            pl.BlockSpec((k, 128), lambda q, i: (0, q)),
        ],
        out_specs=pl.BlockSpec((1, MBLK, 128), lambda q, i: (q, i, 0)),
        out_shape=jax.ShapeDtypeStruct((4, m, 128), jnp.float32),
    )(x, w)


@functools.partial(
    pl.kernel,
    out_type=[
        jax.ShapeDtypeStruct((EP, 16), jnp.float32),
        jax.ShapeDtypeStruct((NC, NP, 16), jnp.float32),
    ],
    mesh=_mesh,
    compiler_params=pltpu.CompilerParams(use_tc_tiling_on_sc=False),
    scratch_types=[
        pltpu.VMEM((ECA,), jnp.int32),
        pltpu.VMEM((ECA,), jnp.int32),
        [pltpu.VMEM((KA, 16), jnp.float32) for _ in range(NBUF)],
        [pltpu.VMEM((KA, 16), jnp.float32) for _ in range(NBUF)],
        [pltpu.VMEM((KA, 16), jnp.float32) for _ in range(NBUF)],
        [pltpu.VMEM((KA,), jnp.int32) for _ in range(NBUF)],
        pltpu.VMEM((16,), jnp.float32),
        pltpu.VMEM_SHARED((NP, 16), jnp.float32),
        [pltpu.SemaphoreType.DMA for _ in range(NBUF)],
        [pltpu.SemaphoreType.DMA for _ in range(NBUF)],
        [pltpu.SemaphoreType.DMA for _ in range(NBUF)],
    ],
)
def _phase_a(a_src_hbm, a_dst_hbm, src_hbm, dst_hbm, shift_hbm, zden_hbm,
             ex_hbm, den_hbm,
             src_v, dst_v, sbufs, dbufs, exbufs, didxs, shift_v, den_sp,
             gsems, wsems, dsems):
    c = lax.axis_index("c")
    s = lax.axis_index("s")
    tile = c * NS + s
    ebase = tile * ECA
    pltpu.sync_copy(src_hbm.at[pl.ds(ebase, ECA)], src_v)
    pltpu.sync_copy(dst_hbm.at[pl.ds(ebase, ECA)], dst_v)
    pltpu.sync_copy(shift_hbm, shift_v)
    r0 = s * RPT
    pltpu.sync_copy(zden_hbm.at[pl.ds(r0, RPT)], den_sp.at[pl.ds(r0, RPT)])

    @pl.when((c == 0) & (s == 0))
    def _():
        pltpu.sync_copy(zden_hbm.at[pl.ds(0, EP - E)], ex_hbm.at[pl.ds(E, EP - E)])

    plsc.subcore_barrier()

    shv = shift_v[...]
    NB = ECA // KA
    NIT = NB // NBUF

    def issue(j, blk):
        base = blk * KA
        pltpu.async_copy(a_src_hbm.at[src_v.at[pl.ds(base, KA)]], sbufs[j], gsems[j])
        pltpu.async_copy(a_dst_hbm.at[dst_v.at[pl.ds(base, KA)]], dbufs[j], gsems[j])

    def wait_g(j, blk):
        base = blk * KA
        pltpu.make_async_copy(a_src_hbm.at[src_v.at[pl.ds(base, KA)]], sbufs[j], gsems[j]).wait()
        pltpu.make_async_copy(a_dst_hbm.at[dst_v.at[pl.ds(base, KA)]], dbufs[j], gsems[j]).wait()

    def wait_s(j, blk):
        base = blk * KA
        pltpu.make_async_copy(exbufs[j], ex_hbm.at[pl.ds(ebase + base, KA)], wsems[j]).wait()
        pltpu.make_async_copy(exbufs[j], den_sp.at[didxs[j]], dsems[j]).wait()

    for j in range(NBUF):
        issue(j, j)

    def mbody(m, carry):
        for j in range(NBUF):
            blk = m * NBUF + j
            wait_g(j, blk)

            @pl.when(m > 0)
            def _(j=j, blk=blk):
                wait_s(j, blk)

            base = blk * KA
            for t in range(KA // 16):
                didxs[j][pl.ds(16 * t, 16)] = dst_v[pl.ds(base + 16 * t, 16)]
            for jj in range(KA):
                al = sbufs[j][jj, :] + dbufs[j][jj, :]
                al = jnp.maximum(al, 0.2 * al)
                exbufs[j][jj, :] = jnp.exp(al - shv)
            pltpu.async_copy(exbufs[j], ex_hbm.at[pl.ds(ebase + base, KA)], wsems[j])
            pltpu.make_async_copy(exbufs[j], den_sp.at[didxs[j]], dsems[j]).start(add=True)

            @pl.when(m < NIT - 1)
            def _(j=j, blk=blk):
                issue(j, blk + NBUF)
        return carry

    lax.fori_loop(0, NIT, mbody, 0)
    for j in range(NBUF):
        wait_s(j, (NIT - 1) * NBUF + j)
    plsc.subcore_barrier()
    pltpu.sync_copy(den_sp.at[pl.ds(r0, RPT)], den_hbm.at[c, pl.ds(r0, RPT)])


@functools.partial(
    pl.kernel,
    out_type=jax.ShapeDtypeStruct((8, NP, 64), jnp.float32),
    mesh=_mesh,
    compiler_params=pltpu.CompilerParams(use_tc_tiling_on_sc=False),
    scratch_types=[
        pltpu.VMEM((CH,), jnp.int32),
        pltpu.VMEM((CH,), jnp.int32),
        [pltpu.VMEM((KB, 64), jnp.float32) for _ in range(NBUF)],
        [pltpu.VMEM((KB, 64), jnp.float32) for _ in range(NBUF)],
        [pltpu.VMEM((KB, 16), jnp.float32) for _ in range(NBUF)],
        [pltpu.VMEM((KB,), jnp.int32) for _ in range(NBUF)],
        pltpu.VMEM_SHARED((NP, 64), jnp.float32),
        [pltpu.SemaphoreType.DMA for _ in range(NBUF)],
        [pltpu.SemaphoreType.DMA for _ in range(NBUF)],
        [pltpu.SemaphoreType.DMA for _ in range(NBUF)],
    ],
)
def _phase_b(h2_hbm, ex_hbm, src_hbm, dst_hbm, zacc_hbm,
             acc_hbm,
             src_v, dst_v, gbufs, obufs, exbufs, didxs, acc_sp, gsems, esems, ssems):
    c = lax.axis_index("c")
    s = lax.axis_index("s")
    r0 = s * RPT
    NB = CH // KB
    NIT = NB // NBUF

    def pass_body(p, carry):
        q = 4 * c + p
        hq = h2_hbm.at[q]
        iq0 = jnp.full((16,), q, jnp.int32)
        pltpu.sync_copy(zacc_hbm.at[pl.ds(r0, RPT)], acc_sp.at[pl.ds(r0, RPT)])
        plsc.subcore_barrier()

        def chunk_body(ch, carry3):
            cbase = s * ECB + ch * CH
            pltpu.sync_copy(src_hbm.at[pl.ds(cbase, CH)], src_v)
            pltpu.sync_copy(dst_hbm.at[pl.ds(cbase, CH)], dst_v)

            def issue(j, blk):
                base = blk * KB
                pltpu.async_copy(hq.at[src_v.at[pl.ds(base, KB)]], gbufs[j], gsems[j])
                pltpu.async_copy(ex_hbm.at[pl.ds(cbase + base, KB)], exbufs[j], esems[j])

            def wait_g(j, blk):
                base = blk * KB
                pltpu.make_async_copy(hq.at[src_v.at[pl.ds(base, KB)]], gbufs[j], gsems[j]).wait()
                pltpu.make_async_copy(ex_hbm.at[pl.ds(cbase + base, KB)], exbufs[j], esems[j]).wait()

            def wait_s(j):
                pltpu.make_async_copy(obufs[j], acc_sp.at[didxs[j]], ssems[j]).wait()

            for j in range(NBUF):
                issue(j, j)

            def mbody(m, carry2):
                for j in range(NBUF):
                    blk = m * NBUF + j
                    wait_g(j, blk)

                    @pl.when(m > 0)
                    def _(j=j):
                        wait_s(j)

                    base = blk * KB
                    for t in range(KB // 16):
                        didxs[j][pl.ds(16 * t, 16)] = dst_v[pl.ds(base + 16 * t, 16)]
                    gb, ob, eb = gbufs[j], obufs[j], exbufs[j]
                    for jj in range(KB):
                        s0 = _take16(eb[jj, :], iq0)
                        for r in range(4):
                            sl = pl.ds(r * 16, 16)
                            ob[jj, sl] = gb[jj, sl] * s0
                    pltpu.make_async_copy(ob, acc_sp.at[didxs[j]], ssems[j]).start(add=True)

                    @pl.when(m < NIT - 1)
                    def _(j=j, blk=blk):
                        issue(j, blk + NBUF)
                return carry2

            lax.fori_loop(0, NIT, mbody, 0)
            for j in range(NBUF):
                wait_s(j)
            return carry3

        lax.fori_loop(0, ECB // CH, chunk_body, 0)
        plsc.subcore_barrier()
        pltpu.sync_copy(acc_sp.at[pl.ds(r0, RPT)], acc_hbm.at[q, pl.ds(r0, RPT)])
        plsc.subcore_barrier()
        return carry

    lax.fori_loop(0, 4, pass_body, 0)


def _gat_layer(xin, src, dst, W, att_src, att_dst, b, zden, zacc):
    n = xin.shape[0]
    h2 = _matmul_h2(xin, W)
    as_r = att_src.reshape(HEADS, HDIM)
    ad_r = att_dst.reshape(HEADS, HDIM)
    Ws = (W.reshape(-1, HEADS, HDIM) * as_r[None]).sum(-1)
    Wd = (W.reshape(-1, HEADS, HDIM) * ad_r[None]).sum(-1)
    Wsd = jnp.concatenate([Ws, Wd, jnp.zeros((W.shape[0], 112), jnp.float32)], axis=1)
    acat = _matmul(xin, Wsd)[:, :16]
    a_src16 = jnp.concatenate([acat[:, :8], jnp.zeros((n, 8), jnp.float32)], axis=1)
    a_dst16 = jnp.concatenate([acat[:, 8:16], jnp.zeros((n, 8), jnp.float32)], axis=1)
    shift8 = jnp.max(acat[:, :8], axis=0) + jnp.max(acat[:, 8:16], axis=0)
    shift16 = jnp.concatenate([shift8, jnp.full((8,), 1e9, jnp.float32)])
    ex16, den_part = _phase_a(a_src16, a_dst16, src, dst, shift16, zden)
    h8 = h2.reshape(4, n, 2, HDIM).transpose(0, 2, 1, 3).reshape(8, n, HDIM)
    srcp = jnp.pad(src, (0, EP - E))
    dstp = jnp.pad(dst, (0, EP - E))
    acc = _phase_b(h8, ex16, srcp, dstp, zacc)
    den = den_part[0, :, :8] + den_part[1, :, :8]
    o = acc / (den.T[:, :, None] + 1e-30)
    o = o + b.reshape(HEADS, 1, HDIM)
    o = jax.nn.relu(o)
    return o.transpose(1, 0, 2).reshape(n, HOUT)


def _pool_kernel(b_ref, h_ref, o_ref):
    i = pl.program_id(0)

    @pl.when(i == 0)
    def _():
        o_ref[...] = jnp.zeros_like(o_ref)

    bvals = b_ref[0, 0, :]
    m = (bvals[None, :] == lax.broadcasted_iota(jnp.int32, (G, MBLK), 0)).astype(jnp.float32)
    o_ref[...] += jnp.dot(m, h_ref[...], preferred_element_type=jnp.float32)


def _pool(batch, h):
    b3 = batch.reshape(NP // MBLK, 1, MBLK)
    return pl.pallas_call(
        _pool_kernel,
        grid=(NP // MBLK,),
        in_specs=[
            pl.BlockSpec((1, 1, MBLK), lambda i: (i, 0, 0)),
            pl.BlockSpec((MBLK, HOUT), lambda i: (i, 0)),
        ],
        out_specs=pl.BlockSpec((G, HOUT), lambda i: (0, 0)),
        out_shape=jax.ShapeDtypeStruct((G, HOUT), jnp.float32),
    )(b3, h)


def kernel(x, edge_index, batch, W0, as0, ad0, b0, W1, as1, ad1, b1, W2, as2, ad2, b2, lin1_W, lin1_b, lin2_W, lin2_b):
    src = edge_index[0]
    dst = edge_index[1]
    zden = jnp.zeros((NP, 16), jnp.float32)
    zacc = jnp.zeros((NP, 64), jnp.float32)
    h = jnp.pad(x, ((0, NP - N), (0, 0)))
    batch_p = jnp.pad(batch, (0, NP - N), constant_values=G)
    for (W, a_s, a_d, b) in [(W0, as0, ad0, b0), (W1, as1, ad1, b1), (W2, as2, ad2, b2)]:
        h = _gat_layer(h, src, dst, W, a_s, a_d, b, zden, zacc)
    pooled = _pool(batch_p, h)
    z = jax.nn.relu(pooled @ lin1_W + lin1_b)
    out = z @ lin2_W + lin2_b
    return out


# speedup vs baseline: 1.9748x; 1.9748x over previous
import functools
import jax
import jax.numpy as jnp
from jax import lax
from jax.experimental import pallas as pl
from jax.experimental.pallas import tpu as pltpu
from jax.experimental.pallas import tpu_sc as plsc

N = 10000
NP = 10240
E = 320000
XDIM = 128
HEADS = 8
HDIM = 64
G = 64
HOUT = 512

NC = 2
NS = 16
MBLK = 640
KA = 80
KB = 32
NBUF = 5
ECA = E // (NC * NS)
ECB = E // NS
RPT = NP // NS

_mesh = plsc.VectorSubcoreMesh(core_axis_name="c", subcore_axis_name="s")

_GDN = lax.GatherDimensionNumbers(
    offset_dims=(), collapsed_slice_dims=(0,), start_index_map=(0,))


def _take16(v, idx):
    return lax.gather(v, idx[:, None], _GDN, (1,),
                      mode=lax.GatherScatterMode.PROMISE_IN_BOUNDS)


def _mm_kernel(x_ref, w_ref, o_ref):
    o_ref[...] = jnp.dot(x_ref[...], w_ref[...], preferred_element_type=jnp.float32)


def _matmul(x, w):
    m, k = x.shape
    _, n = w.shape
    return pl.pallas_call(
        _mm_kernel,
        grid=(m // MBLK,),
        in_specs=[
            pl.BlockSpec((MBLK, k), lambda i: (i, 0)),
            pl.BlockSpec((k, n), lambda i: (0, 0)),
        ],
        out_specs=pl.BlockSpec((MBLK, n), lambda i: (i, 0)),
        out_shape=jax.ShapeDtypeStruct((m, n), jnp.float32),
    )(x, w)


def _mm_h2_kernel(x_ref, w_ref, o_ref):
    o_ref[0] = jnp.dot(x_ref[...], w_ref[...], preferred_element_type=jnp.float32)


def _matmul_h2(x, w):
    m, k = x.shape
    return pl.pallas_call(
        _mm_h2_kernel,
        grid=(4, m // MBLK),
        in_specs=[
            pl.BlockSpec((MBLK, k), lambda q, i: (i, 0)),
            pl.BlockSpec((k, 128), lambda q, i: (0, q)),
        ],
        out_specs=pl.BlockSpec((1, MBLK, 128), lambda q, i: (q, i, 0)),
        out_shape=jax.ShapeDtypeStruct((4, m, 128), jnp.float32),
    )(x, w)


@functools.partial(
    pl.kernel,
    out_type=[
        jax.ShapeDtypeStruct((E, 16), jnp.float32),
        jax.ShapeDtypeStruct((NC, NP, 16), jnp.float32),
    ],
    mesh=_mesh,
    compiler_params=pltpu.CompilerParams(use_tc_tiling_on_sc=False),
    scratch_types=[
        pltpu.VMEM((ECA,), jnp.int32),
        pltpu.VMEM((ECA,), jnp.int32),
        [pltpu.VMEM((KA, 16), jnp.float32) for _ in range(NBUF)],
        [pltpu.VMEM((KA, 16), jnp.float32) for _ in range(NBUF)],
        [pltpu.VMEM((KA, 16), jnp.float32) for _ in range(NBUF)],
        [pltpu.VMEM((KA,), jnp.int32) for _ in range(NBUF)],
        pltpu.VMEM((16,), jnp.float32),
        pltpu.VMEM_SHARED((NP, 16), jnp.float32),
        [pltpu.SemaphoreType.DMA for _ in range(NBUF)],
        [pltpu.SemaphoreType.DMA for _ in range(NBUF)],
        [pltpu.SemaphoreType.DMA for _ in range(NBUF)],
    ],
)
def _phase_a(a_src_hbm, a_dst_hbm, src_hbm, dst_hbm, shift_hbm, zden_hbm,
             ex_hbm, den_hbm,
             src_v, dst_v, sbufs, dbufs, exbufs, didxs, shift_v, den_sp,
             gsems, wsems, dsems):
    c = lax.axis_index("c")
    s = lax.axis_index("s")
    tile = c * NS + s
    ebase = tile * ECA
    pltpu.sync_copy(src_hbm.at[pl.ds(ebase, ECA)], src_v)
    pltpu.sync_copy(dst_hbm.at[pl.ds(ebase, ECA)], dst_v)
    pltpu.sync_copy(shift_hbm, shift_v)
    r0 = s * RPT
    pltpu.sync_copy(zden_hbm.at[pl.ds(r0, RPT)], den_sp.at[pl.ds(r0, RPT)])
    plsc.subcore_barrier()

    shv = shift_v[...]
    NB = ECA // KA
    NIT = NB // NBUF

    def issue(j, blk):
        base = blk * KA
        pltpu.async_copy(a_src_hbm.at[src_v.at[pl.ds(base, KA)]], sbufs[j], gsems[j])
        pltpu.async_copy(a_dst_hbm.at[dst_v.at[pl.ds(base, KA)]], dbufs[j], gsems[j])

    def wait_g(j, blk):
        base = blk * KA
        pltpu.make_async_copy(a_src_hbm.at[src_v.at[pl.ds(base, KA)]], sbufs[j], gsems[j]).wait()
        pltpu.make_async_copy(a_dst_hbm.at[dst_v.at[pl.ds(base, KA)]], dbufs[j], gsems[j]).wait()

    def wait_s(j, blk):
        base = blk * KA
        pltpu.make_async_copy(exbufs[j], ex_hbm.at[pl.ds(ebase + base, KA)], wsems[j]).wait()
        pltpu.make_async_copy(exbufs[j], den_sp.at[didxs[j]], dsems[j]).wait()

    for j in range(NBUF):
        issue(j, j)

    def mbody(m, carry):
        for j in range(NBUF):
            blk = m * NBUF + j
            wait_g(j, blk)

            @pl.when(m > 0)
            def _(j=j, blk=blk):
                wait_s(j, blk)

            base = blk * KA
            for t in range(KA // 16):
                didxs[j][pl.ds(16 * t, 16)] = dst_v[pl.ds(base + 16 * t, 16)]
            for jj in range(KA):
                al = sbufs[j][jj, :] + dbufs[j][jj, :]
                al = jnp.maximum(al, 0.2 * al)
                exbufs[j][jj, :] = jnp.exp(al - shv)
            pltpu.async_copy(exbufs[j], ex_hbm.at[pl.ds(ebase + base, KA)], wsems[j])
            pltpu.make_async_copy(exbufs[j], den_sp.at[didxs[j]], dsems[j]).start(add=True)

            @pl.when(m < NIT - 1)
            def _(j=j, blk=blk):
                issue(j, blk + NBUF)
        return carry

    lax.fori_loop(0, NIT, mbody, 0)
    for j in range(NBUF):
        wait_s(j, (NIT - 1) * NBUF + j)
    plsc.subcore_barrier()
    pltpu.sync_copy(den_sp.at[pl.ds(r0, RPT)], den_hbm.at[c, pl.ds(r0, RPT)])


@functools.partial(
    pl.kernel,
    out_type=jax.ShapeDtypeStruct((8, NP, 64), jnp.float32),
    mesh=_mesh,
    compiler_params=pltpu.CompilerParams(use_tc_tiling_on_sc=False),
    scratch_types=[
        pltpu.VMEM((ECB,), jnp.int32),
        pltpu.VMEM((ECB,), jnp.int32),
        [pltpu.VMEM((KB, 64), jnp.float32) for _ in range(NBUF)],
        [pltpu.VMEM((KB, 64), jnp.float32) for _ in range(NBUF)],
        [pltpu.VMEM((KB, 16), jnp.float32) for _ in range(NBUF)],
        [pltpu.VMEM((KB,), jnp.int32) for _ in range(NBUF)],
        pltpu.VMEM_SHARED((NP, 64), jnp.float32),
        [pltpu.SemaphoreType.DMA for _ in range(NBUF)],
        [pltpu.SemaphoreType.DMA for _ in range(NBUF)],
        [pltpu.SemaphoreType.DMA for _ in range(NBUF)],
    ],
)
def _phase_b(h2_hbm, ex_hbm, src_hbm, dst_hbm, zacc_hbm,
             acc_hbm,
             src_v, dst_v, gbufs, obufs, exbufs, didxs, acc_sp, gsems, esems, ssems):
    c = lax.axis_index("c")
    s = lax.axis_index("s")
    ebase = s * ECB
    pltpu.sync_copy(src_hbm.at[pl.ds(ebase, ECB)], src_v)
    pltpu.sync_copy(dst_hbm.at[pl.ds(ebase, ECB)], dst_v)
    r0 = s * RPT
    NB = ECB // KB
    NIT = NB // NBUF

    def pass_body(p, carry):
        q = 4 * c + p
        hq = h2_hbm.at[q]
        iq0 = jnp.full((16,), q, jnp.int32)
        pltpu.sync_copy(zacc_hbm.at[pl.ds(r0, RPT)], acc_sp.at[pl.ds(r0, RPT)])
        plsc.subcore_barrier()

        def issue(j, blk):
            base = blk * KB
            pltpu.async_copy(hq.at[src_v.at[pl.ds(base, KB)]], gbufs[j], gsems[j])
            pltpu.async_copy(ex_hbm.at[pl.ds(ebase + base, KB)], exbufs[j], esems[j])

        def wait_g(j, blk):
            base = blk * KB
            pltpu.make_async_copy(hq.at[src_v.at[pl.ds(base, KB)]], gbufs[j], gsems[j]).wait()
            pltpu.make_async_copy(ex_hbm.at[pl.ds(ebase + base, KB)], exbufs[j], esems[j]).wait()

        def wait_s(j):
            pltpu.make_async_copy(obufs[j], acc_sp.at[didxs[j]], ssems[j]).wait()

        for j in range(NBUF):
            issue(j, j)

        def mbody(m, carry2):
            for j in range(NBUF):
                blk = m * NBUF + j
                wait_g(j, blk)

                @pl.when(m > 0)
                def _(j=j):
                    wait_s(j)

                base = blk * KB
                for t in range(KB // 16):
                    didxs[j][pl.ds(16 * t, 16)] = dst_v[pl.ds(base + 16 * t, 16)]
                gb, ob, eb = gbufs[j], obufs[j], exbufs[j]
                for jj in range(KB):
                    s0 = _take16(eb[jj, :], iq0)
                    for r in range(4):
                        sl = pl.ds(r * 16, 16)
                        ob[jj, sl] = gb[jj, sl] * s0
                pltpu.make_async_copy(ob, acc_sp.at[didxs[j]], ssems[j]).start(add=True)

                @pl.when(m < NIT - 1)
                def _(j=j, blk=blk):
                    issue(j, blk + NBUF)
            return carry2

        lax.fori_loop(0, NIT, mbody, 0)
        for j in range(NBUF):
            wait_s(j)
        plsc.subcore_barrier()
        pltpu.sync_copy(acc_sp.at[pl.ds(r0, RPT)], acc_hbm.at[q, pl.ds(r0, RPT)])
        plsc.subcore_barrier()
        return carry

    lax.fori_loop(0, 4, pass_body, 0)


def _gat_layer(xin, src, dst, W, att_src, att_dst, b, zden, zacc):
    n = xin.shape[0]
    h2 = _matmul_h2(xin, W)
    as_r = att_src.reshape(HEADS, HDIM)
    ad_r = att_dst.reshape(HEADS, HDIM)
    Ws = (W.reshape(-1, HEADS, HDIM) * as_r[None]).sum(-1)
    Wd = (W.reshape(-1, HEADS, HDIM) * ad_r[None]).sum(-1)
    Wsd = jnp.concatenate([Ws, Wd, jnp.zeros((W.shape[0], 112), jnp.float32)], axis=1)
    acat = _matmul(xin, Wsd)[:, :16]
    a_src16 = jnp.concatenate([acat[:, :8], jnp.zeros((n, 8), jnp.float32)], axis=1)
    a_dst16 = jnp.concatenate([acat[:, 8:16], jnp.zeros((n, 8), jnp.float32)], axis=1)
    shift8 = jnp.max(acat[:, :8], axis=0) + jnp.max(acat[:, 8:16], axis=0)
    shift16 = jnp.concatenate([shift8, jnp.full((8,), 1e9, jnp.float32)])
    ex16, den_part = _phase_a(a_src16, a_dst16, src, dst, shift16, zden)
    h8 = h2.reshape(4, n, 2, HDIM).transpose(0, 2, 1, 3).reshape(8, n, HDIM)
    acc = _phase_b(h8, ex16, src, dst, zacc)
    den = den_part[0, :, :8] + den_part[1, :, :8]
    o = acc / (den.T[:, :, None] + 1e-30)
    o = o + b.reshape(HEADS, 1, HDIM)
    o = jax.nn.relu(o)
    return o.transpose(1, 0, 2).reshape(n, HOUT)


def _pool_kernel(b_ref, h_ref, o_ref):
    i = pl.program_id(0)

    @pl.when(i == 0)
    def _():
        o_ref[...] = jnp.zeros_like(o_ref)

    bvals = b_ref[0, 0, :]
    m = (bvals[None, :] == lax.broadcasted_iota(jnp.int32, (G, MBLK), 0)).astype(jnp.float32)
    o_ref[...] += jnp.dot(m, h_ref[...], preferred_element_type=jnp.float32)


def _pool(batch, h):
    b3 = batch.reshape(NP // MBLK, 1, MBLK)
    return pl.pallas_call(
        _pool_kernel,
        grid=(NP // MBLK,),
        in_specs=[
            pl.BlockSpec((1, 1, MBLK), lambda i: (i, 0, 0)),
            pl.BlockSpec((MBLK, HOUT), lambda i: (i, 0)),
        ],
        out_specs=pl.BlockSpec((G, HOUT), lambda i: (0, 0)),
        out_shape=jax.ShapeDtypeStruct((G, HOUT), jnp.float32),
    )(b3, h)


def kernel(x, edge_index, batch, W0, as0, ad0, b0, W1, as1, ad1, b1, W2, as2, ad2, b2, lin1_W, lin1_b, lin2_W, lin2_b):
    src = edge_index[0]
    dst = edge_index[1]
    zden = jnp.zeros((NP, 16), jnp.float32)
    zacc = jnp.zeros((NP, 64), jnp.float32)
    h = jnp.pad(x, ((0, NP - N), (0, 0)))
    batch_p = jnp.pad(batch, (0, NP - N), constant_values=G)
    for (W, a_s, a_d, b) in [(W0, as0, ad0, b0), (W1, as1, ad1, b1), (W2, as2, ad2, b2)]:
        h = _gat_layer(h, src, dst, W, a_s, a_d, b, zden, zacc)
    pooled = _pool(batch_p, h)
    z = jax.nn.relu(pooled @ lin1_W + lin1_b)
    out = z @ lin2_W + lin2_b
    return out
